# baseline (speedup 1.0000x reference)
import functools

import jax
import jax.numpy as jnp
from jax import lax
from jax.experimental import pallas as pl
from jax.experimental.pallas import tpu as pltpu
from jax.experimental.pallas import tpu_sc as plsc

N = 10000
NP = 10240
E = 320000
F = 128
H = 64
HS = 32
CP = 16
CHUNK = 128
NC = 2
NS = 16
NW = NC * NS
NBUF = 8
NGRP = 10
CPT = NBUF * NGRP
EP = NW * CPT * CHUNK
RPT = NP // NS
BN = 1000
NMAIN = E // CHUNK
NPADC = CPT * NW - NMAIN
NTAIL = CPT - NPADC

_mesh = plsc.VectorSubcoreMesh(core_axis_name="c", subcore_axis_name="s")


def _load_idx(main_hbm, pad_hbm, buf, w):
    @pl.when(w < NW - 1)
    def _():
        pltpu.sync_copy(main_hbm.at[pl.ds(w * CPT, CPT)], buf)

    @pl.when(w == NW - 1)
    def _():
        pltpu.sync_copy(main_hbm.at[pl.ds(NMAIN - NTAIL, NTAIL)], buf.at[pl.ds(0, NTAIL)])
        pltpu.sync_copy(pad_hbm, buf.at[pl.ds(NTAIL, NPADC)])


def _deg_body(dst_hbm, pdst_hbm, zeros_hbm, ones_hbm, out_hbm, acc, dstb, ones_v, ssem):
    c = lax.axis_index("c")
    s = lax.axis_index("s")
    w = c * NS + s
    pltpu.sync_copy(zeros_hbm.at[pl.ds(s * RPT, RPT)], acc.at[pl.ds(s * RPT, RPT)])
    pltpu.sync_copy(ones_hbm, ones_v)
    _load_idx(dst_hbm, pdst_hbm, dstb, w)
    plsc.subcore_barrier()

    def grp(g, carry):
        for b in range(NBUF):
            pltpu.async_copy(ones_v, acc.at[dstb.at[g * NBUF + b]], ssem.at[b], add=True)
        for b in range(NBUF):
            pltpu.make_async_copy(ones_v, acc.at[dstb.at[g * NBUF + b]], ssem.at[b]).wait()
        return carry

    lax.fori_loop(0, NGRP, grp, 0)
    plsc.subcore_barrier()
    pltpu.sync_copy(acc.at[pl.ds(s * RPT, RPT)], out_hbm.at[c].at[pl.ds(s * RPT, RPT)])


def _agg_body(nsplit, d, h_hbm, src_hbm, dst_hbm, psrc_hbm, pdst_hbm, zeros_hbm,
              out_hbm, acc, hstage, srcb, dstb, rows, gsem, ssem):
    c = lax.axis_index("c")
    s = lax.axis_index("s")
    w = c * NS + s
    _load_idx(src_hbm, psrc_hbm, srcb, w)
    _load_idx(dst_hbm, pdst_hbm, dstb, w)

    for kp in range(nsplit):
        @pl.when(c == 0)
        def _():
            pltpu.sync_copy(h_hbm.at[kp].at[pl.ds(s * RPT, RPT)], acc.at[pl.ds(s * RPT, RPT)])

        @pl.when(c != 0)
        def _():
            pltpu.sync_copy(zeros_hbm.at[pl.ds(s * RPT, RPT)], acc.at[pl.ds(s * RPT, RPT)])

        pltpu.sync_copy(h_hbm.at[kp].at[pl.ds(s * RPT, RPT)], hstage.at[pl.ds(s * RPT, RPT)])
        plsc.subcore_barrier()

        for b in range(NBUF):
            pltpu.async_copy(hstage.at[srcb.at[b]], rows.at[b], gsem.at[b])

        def grp(g, carry):
            for b in range(NBUF):
                j = g * NBUF + b
                pltpu.make_async_copy(hstage.at[srcb.at[j]], rows.at[b], gsem.at[b]).wait()
                pltpu.async_copy(rows.at[b], acc.at[dstb.at[j]], ssem.at[b], add=True)
            for b in range(NBUF):
                j = g * NBUF + b
                pltpu.make_async_copy(rows.at[b], acc.at[dstb.at[j]], ssem.at[b]).wait()
                pltpu.async_copy(hstage.at[srcb.at[j + NBUF]], rows.at[b], gsem.at[b])
            return carry

        lax.fori_loop(0, NGRP - 1, grp, 0)
        for b in range(NBUF):
            j = (NGRP - 1) * NBUF + b
            pltpu.make_async_copy(hstage.at[srcb.at[j]], rows.at[b], gsem.at[b]).wait()
            pltpu.async_copy(rows.at[b], acc.at[dstb.at[j]], ssem.at[b], add=True)
        for b in range(NBUF):
            j = (NGRP - 1) * NBUF + b
            pltpu.make_async_copy(rows.at[b], acc.at[dstb.at[j]], ssem.at[b]).wait()
        plsc.subcore_barrier()
        pltpu.sync_copy(acc.at[pl.ds(s * RPT, RPT)],
                        out_hbm.at[c].at[kp].at[pl.ds(s * RPT, RPT)])


def _sc_degree(dst2d, pdst, zeros1, ones1):
    return pl.kernel(
        _deg_body,
        out_type=jax.ShapeDtypeStruct((NC, NP, CP), jnp.float32),
        mesh=_mesh,
        scratch_types=[
            pltpu.VMEM_SHARED((NP, CP), jnp.float32),
            pltpu.VMEM((CPT, CHUNK), jnp.int32),
            pltpu.VMEM((CHUNK, CP), jnp.float32),
            pltpu.SemaphoreType.DMA((NBUF,)),
        ],
        compiler_params=pltpu.CompilerParams(use_tc_tiling_on_sc=False),
    )(dst2d, pdst, zeros1, ones1)


def _sc_aggregate(nsplit, d, h, src2d, dst2d, psrc, pdst, zerosd):
    body = functools.partial(_agg_body, nsplit, d)
    return pl.kernel(
        body,
        out_type=jax.ShapeDtypeStruct((NC, nsplit, NP, d), jnp.float32),
        mesh=_mesh,
        scratch_types=[
            pltpu.VMEM_SHARED((NP, d), jnp.float32),
            pltpu.VMEM_SHARED((NP, d), jnp.float32),
            pltpu.VMEM((CPT, CHUNK), jnp.int32),
            pltpu.VMEM((CPT, CHUNK), jnp.int32),
            pltpu.VMEM((NBUF, CHUNK, d), jnp.float32),
            pltpu.SemaphoreType.DMA((NBUF,)),
            pltpu.SemaphoreType.DMA((NBUF,)),
        ],
        compiler_params=pltpu.CompilerParams(use_tc_tiling_on_sc=False),
    )(h, src2d, dst2d, psrc, pdst, zerosd)


def _tc_pre_body(x_ref, w1_ref, degp_ref, h1s_ref, dis_ref):
    deg = degp_ref[0, :, 0:1] + degp_ref[1, :, 0:1] + 1.0
    dis = lax.rsqrt(deg)
    dis_ref[...] = dis
    res = jnp.dot(x_ref[...], w1_ref[...], preferred_element_type=jnp.float32, precision=lax.Precision.HIGHEST) * dis
    h1s_ref[0] = res[:, :HS]
    h1s_ref[1] = res[:, HS:]


def _tc_mid_body(agg_ref, dis_ref, b1_ref, w2a_ref, w2b_ref, h2s_ref):
    dis = dis_ref[...]
    yl = agg_ref[0, 0] + agg_ref[1, 0]
    yr = agg_ref[0, 1] + agg_ref[1, 1]
    rl = jnp.maximum(dis * yl + b1_ref[:, :HS], 0.0)
    rr = jnp.maximum(dis * yr + b1_ref[:, HS:], 0.0)
    h2 = (jnp.dot(rl, w2a_ref[...], preferred_element_type=jnp.float32, precision=lax.Precision.HIGHEST)
          + jnp.dot(rr, w2b_ref[...], preferred_element_type=jnp.float32, precision=lax.Precision.HIGHEST))
    h2s_ref[...] = h2 * dis


def _tc_post_body(agg_ref, dis_ref, b2_ref, out_ref):
    y2 = agg_ref[0, 0] + agg_ref[1, 0]
    out_ref[...] = dis_ref[...] * y2[:, : out_ref.shape[1]] + b2_ref[...]


def kernel(x, edge_index, W1, b1, W2, b2):
    src2d = edge_index[0].astype(jnp.int32).reshape(NMAIN, CHUNK)
    dst2d = edge_index[1].astype(jnp.int32).reshape(NMAIN, CHUNK)
    psrc = jnp.zeros((NPADC, CHUNK), jnp.int32)
    pdst = N + jnp.arange(NPADC * CHUNK, dtype=jnp.int32).reshape(NPADC, CHUNK) % (NP - N)
    zeros1 = jnp.zeros((NP, CP), jnp.float32)
    ones1 = jnp.ones((CHUNK, CP), jnp.float32)
    zeros32 = jnp.zeros((NP, HS), jnp.float32)
    zeros16 = jnp.zeros((NP, CP), jnp.float32)
    W2p = jnp.concatenate([W2, jnp.zeros((H, CP - W2.shape[1]), jnp.float32)], axis=1)
    b1r = b1.reshape(1, H)
    b2r = b2.reshape(1, -1)
    nb = N // BN

    degp = _sc_degree(dst2d, pdst, zeros1, ones1)

    h1s, dis = pl.pallas_call(
        _tc_pre_body,
        grid=(nb,),
        in_specs=[
            pl.BlockSpec((BN, F), lambda i: (i, 0)),
            pl.BlockSpec((F, H), lambda i: (0, 0)),
            pl.BlockSpec((NC, BN, CP), lambda i: (0, i, 0)),
        ],
        out_specs=[
            pl.BlockSpec((2, BN, HS), lambda i: (0, i, 0)),
            pl.BlockSpec((BN, 1), lambda i: (i, 0)),
        ],
        out_shape=[
            jax.ShapeDtypeStruct((2, NP, HS), jnp.float32),
            jax.ShapeDtypeStruct((N, 1), jnp.float32),
        ],
    )(x, W1, degp)

    agg1 = _sc_aggregate(2, HS, h1s, src2d, dst2d, psrc, pdst, zeros32)

    h2s = pl.pallas_call(
        _tc_mid_body,
        grid=(nb,),
        in_specs=[
            pl.BlockSpec((NC, 2, BN, HS), lambda i: (0, 0, i, 0)),
            pl.BlockSpec((BN, 1), lambda i: (i, 0)),
            pl.BlockSpec((1, H), lambda i: (0, 0)),
            pl.BlockSpec((HS, CP), lambda i: (0, 0)),
            pl.BlockSpec((HS, CP), lambda i: (0, 0)),
        ],
        out_specs=pl.BlockSpec((BN, CP), lambda i: (i, 0)),
        out_shape=jax.ShapeDtypeStruct((NP, CP), jnp.float32),
    )(agg1, dis, b1r, W2p[:HS], W2p[HS:])

    agg2 = _sc_aggregate(1, CP, h2s.reshape(1, NP, CP), src2d, dst2d, psrc, pdst, zeros16)

    out = pl.pallas_call(
        _tc_post_body,
        grid=(nb,),
        in_specs=[
            pl.BlockSpec((NC, 1, BN, CP), lambda i: (0, 0, i, 0)),
            pl.BlockSpec((BN, 1), lambda i: (i, 0)),
            pl.BlockSpec((1, b2.shape[0]), lambda i: (0, 0)),
        ],
        out_specs=pl.BlockSpec((BN, b2.shape[0]), lambda i: (i, 0)),
        out_shape=jax.ShapeDtypeStruct((N, b2.shape[0]), jnp.float32),
    )(agg2, dis, b2r)

    return out


# speedup vs baseline: 1.0018x; 1.0018x over previous
import functools

import jax
import jax.numpy as jnp
from jax import lax
from jax.experimental import pallas as pl
from jax.experimental.pallas import tpu as pltpu
from jax.experimental.pallas import tpu_sc as plsc

N = 10000
NP = 10240
E = 320000
F = 128
H = 64
HS = 32
CP = 16
CHUNK = 128
NC = 2
NS = 16
NW = NC * NS
NBUF = 8
NGRP = 10
CPT = NBUF * NGRP
EP = NW * CPT * CHUNK
RPT = NP // NS
BN = 1000
NMAIN = E // CHUNK
NPADC = CPT * NW - NMAIN
NTAIL = CPT - NPADC

_mesh = plsc.VectorSubcoreMesh(core_axis_name="c", subcore_axis_name="s")


def _load_idx(main_hbm, pad_hbm, buf, w):
    @pl.when(w < NW - 1)
    def _():
        pltpu.sync_copy(main_hbm.at[pl.ds(w * CPT, CPT)], buf)

    @pl.when(w == NW - 1)
    def _():
        pltpu.sync_copy(main_hbm.at[pl.ds(NMAIN - NTAIL, NTAIL)], buf.at[pl.ds(0, NTAIL)])
        pltpu.sync_copy(pad_hbm, buf.at[pl.ds(NTAIL, NPADC)])


def _deg_body(dst_hbm, pdst_hbm, zeros_hbm, ones_hbm, out_hbm, acc, dstb, ones_v, ssem):
    c = lax.axis_index("c")
    s = lax.axis_index("s")
    w = c * NS + s
    pltpu.sync_copy(zeros_hbm.at[pl.ds(s * RPT, RPT)], acc.at[pl.ds(s * RPT, RPT)])
    pltpu.sync_copy(ones_hbm, ones_v)
    _load_idx(dst_hbm, pdst_hbm, dstb, w)
    plsc.subcore_barrier()

    def grp(g, carry):
        for b in range(NBUF):
            pltpu.async_copy(ones_v, acc.at[dstb.at[g * NBUF + b]], ssem.at[b], add=True)
        for b in range(NBUF):
            pltpu.make_async_copy(ones_v, acc.at[dstb.at[g * NBUF + b]], ssem.at[b]).wait()
        return carry

    lax.fori_loop(0, NGRP, grp, 0)
    plsc.subcore_barrier()
    pltpu.sync_copy(acc.at[pl.ds(s * RPT, RPT)], out_hbm.at[c].at[pl.ds(s * RPT, RPT)])


def _agg_body(nsplit, d, h_hbm, src_hbm, dst_hbm, psrc_hbm, pdst_hbm, zeros_hbm,
              out_hbm, acc, hstage, srcb, dstb, rows, gsem, ssem):
    c = lax.axis_index("c")
    s = lax.axis_index("s")
    w = c * NS + s
    _load_idx(src_hbm, psrc_hbm, srcb, w)
    _load_idx(dst_hbm, pdst_hbm, dstb, w)

    for kp in range(nsplit):
        @pl.when(c == 0)
        def _():
            pltpu.sync_copy(h_hbm.at[kp].at[pl.ds(s * RPT, RPT)], acc.at[pl.ds(s * RPT, RPT)])

        @pl.when(c != 0)
        def _():
            pltpu.sync_copy(zeros_hbm.at[pl.ds(s * RPT, RPT)], acc.at[pl.ds(s * RPT, RPT)])

        pltpu.sync_copy(h_hbm.at[kp].at[pl.ds(s * RPT, RPT)], hstage.at[pl.ds(s * RPT, RPT)])
        plsc.subcore_barrier()

        for b in range(NBUF):
            pltpu.async_copy(hstage.at[srcb.at[b]], rows.at[b], gsem.at[b])

        def grp(g, carry):
            for b in range(NBUF):
                j = g * NBUF + b
                pltpu.make_async_copy(hstage.at[srcb.at[j]], rows.at[b], gsem.at[b]).wait()
                pltpu.async_copy(rows.at[b], acc.at[dstb.at[j]], ssem.at[b], add=True)
            for b in range(NBUF):
                j = g * NBUF + b
                pltpu.make_async_copy(rows.at[b], acc.at[dstb.at[j]], ssem.at[b]).wait()
                pltpu.async_copy(hstage.at[srcb.at[j + NBUF]], rows.at[b], gsem.at[b])
            return carry

        lax.fori_loop(0, NGRP - 1, grp, 0)
        for b in range(NBUF):
            j = (NGRP - 1) * NBUF + b
            pltpu.make_async_copy(hstage.at[srcb.at[j]], rows.at[b], gsem.at[b]).wait()
            pltpu.async_copy(rows.at[b], acc.at[dstb.at[j]], ssem.at[b], add=True)
        for b in range(NBUF):
            j = (NGRP - 1) * NBUF + b
            pltpu.make_async_copy(rows.at[b], acc.at[dstb.at[j]], ssem.at[b]).wait()
        plsc.subcore_barrier()
        pltpu.sync_copy(acc.at[pl.ds(s * RPT, RPT)],
                        out_hbm.at[c].at[kp].at[pl.ds(s * RPT, RPT)])


def _sc_degree(dst2d, pdst, zeros1, ones1):
    return pl.kernel(
        _deg_body,
        out_type=jax.ShapeDtypeStruct((NC, NP, CP), jnp.float32),
        mesh=_mesh,
        scratch_types=[
            pltpu.VMEM_SHARED((NP, CP), jnp.float32),
            pltpu.VMEM((CPT, CHUNK), jnp.int32),
            pltpu.VMEM((CHUNK, CP), jnp.float32),
            pltpu.SemaphoreType.DMA((NBUF,)),
        ],
        compiler_params=pltpu.CompilerParams(use_tc_tiling_on_sc=False),
    )(dst2d, pdst, zeros1, ones1)


def _sc_aggregate(nsplit, d, h, src2d, dst2d, psrc, pdst, zerosd):
    body = functools.partial(_agg_body, nsplit, d)
    return pl.kernel(
        body,
        out_type=jax.ShapeDtypeStruct((NC, nsplit, NP, d), jnp.float32),
        mesh=_mesh,
        scratch_types=[
            pltpu.VMEM_SHARED((NP, d), jnp.float32),
            pltpu.VMEM_SHARED((NP, d), jnp.float32),
            pltpu.VMEM((CPT, CHUNK), jnp.int32),
            pltpu.VMEM((CPT, CHUNK), jnp.int32),
            pltpu.VMEM((NBUF, CHUNK, d), jnp.float32),
            pltpu.SemaphoreType.DMA((NBUF,)),
            pltpu.SemaphoreType.DMA((NBUF,)),
        ],
        compiler_params=pltpu.CompilerParams(use_tc_tiling_on_sc=False),
    )(h, src2d, dst2d, psrc, pdst, zerosd)


def _tc_mm_body(x_ref, w1_ref, h1_ref):
    res = jnp.dot(x_ref[...], w1_ref[...], preferred_element_type=jnp.float32, precision=lax.Precision.HIGHEST)
    h1_ref[0] = res[:, :HS]
    h1_ref[1] = res[:, HS:]


def _tc_scale_body(h1_ref, degp_ref, h1s_ref, dis_ref):
    deg = degp_ref[0, :, 0:1] + degp_ref[1, :, 0:1] + 1.0
    dis = lax.rsqrt(deg)
    dis_ref[...] = dis
    h1s_ref[0] = h1_ref[0] * dis
    h1s_ref[1] = h1_ref[1] * dis


def _tc_mid_body(agg_ref, dis_ref, b1_ref, w2a_ref, w2b_ref, h2s_ref):
    dis = dis_ref[...]
    yl = agg_ref[0, 0] + agg_ref[1, 0]
    yr = agg_ref[0, 1] + agg_ref[1, 1]
    rl = jnp.maximum(dis * yl + b1_ref[:, :HS], 0.0)
    rr = jnp.maximum(dis * yr + b1_ref[:, HS:], 0.0)
    h2 = (jnp.dot(rl, w2a_ref[...], preferred_element_type=jnp.float32, precision=lax.Precision.HIGHEST)
          + jnp.dot(rr, w2b_ref[...], preferred_element_type=jnp.float32, precision=lax.Precision.HIGHEST))
    h2s_ref[...] = h2 * dis


def _tc_post_body(agg_ref, dis_ref, b2_ref, out_ref):
    y2 = agg_ref[0, 0] + agg_ref[1, 0]
    out_ref[...] = dis_ref[...] * y2[:, : out_ref.shape[1]] + b2_ref[...]


def kernel(x, edge_index, W1, b1, W2, b2):
    src2d = edge_index[0].astype(jnp.int32).reshape(NMAIN, CHUNK)
    dst2d = edge_index[1].astype(jnp.int32).reshape(NMAIN, CHUNK)
    psrc = jnp.zeros((NPADC, CHUNK), jnp.int32)
    pdst = N + jnp.arange(NPADC * CHUNK, dtype=jnp.int32).reshape(NPADC, CHUNK) % (NP - N)
    zeros1 = jnp.zeros((NP, CP), jnp.float32)
    ones1 = jnp.ones((CHUNK, CP), jnp.float32)
    zeros32 = jnp.zeros((NP, HS), jnp.float32)
    zeros16 = jnp.zeros((NP, CP), jnp.float32)
    W2p = jnp.concatenate([W2, jnp.zeros((H, CP - W2.shape[1]), jnp.float32)], axis=1)
    b1r = b1.reshape(1, H)
    b2r = b2.reshape(1, -1)
    nb = N // BN

    degp = _sc_degree(dst2d, pdst, zeros1, ones1)

    h1 = pl.pallas_call(
        _tc_mm_body,
        grid=(nb,),
        in_specs=[
            pl.BlockSpec((BN, F), lambda i: (i, 0)),
            pl.BlockSpec((F, H), lambda i: (0, 0)),
        ],
        out_specs=pl.BlockSpec((2, BN, HS), lambda i: (0, i, 0)),
        out_shape=jax.ShapeDtypeStruct((2, NP, HS), jnp.float32),
    )(x, W1)

    h1s, dis = pl.pallas_call(
        _tc_scale_body,
        grid=(nb,),
        in_specs=[
            pl.BlockSpec((2, BN, HS), lambda i: (0, i, 0)),
            pl.BlockSpec((NC, BN, CP), lambda i: (0, i, 0)),
        ],
        out_specs=[
            pl.BlockSpec((2, BN, HS), lambda i: (0, i, 0)),
            pl.BlockSpec((BN, 1), lambda i: (i, 0)),
        ],
        out_shape=[
            jax.ShapeDtypeStruct((2, NP, HS), jnp.float32),
            jax.ShapeDtypeStruct((N, 1), jnp.float32),
        ],
    )(h1, degp)

    agg1 = _sc_aggregate(2, HS, h1s, src2d, dst2d, psrc, pdst, zeros32)

    h2s = pl.pallas_call(
        _tc_mid_body,
        grid=(nb,),
        in_specs=[
            pl.BlockSpec((NC, 2, BN, HS), lambda i: (0, 0, i, 0)),
            pl.BlockSpec((BN, 1), lambda i: (i, 0)),
            pl.BlockSpec((1, H), lambda i: (0, 0)),
            pl.BlockSpec((HS, CP), lambda i: (0, 0)),
            pl.BlockSpec((HS, CP), lambda i: (0, 0)),
        ],
        out_specs=pl.BlockSpec((BN, CP), lambda i: (i, 0)),
        out_shape=jax.ShapeDtypeStruct((NP, CP), jnp.float32),
    )(agg1, dis, b1r, W2p[:HS], W2p[HS:])

    agg2 = _sc_aggregate(1, CP, h2s.reshape(1, NP, CP), src2d, dst2d, psrc, pdst, zeros16)

    out = pl.pallas_call(
        _tc_post_body,
        grid=(nb,),
        in_specs=[
            pl.BlockSpec((NC, 1, BN, CP), lambda i: (0, 0, i, 0)),
            pl.BlockSpec((BN, 1), lambda i: (i, 0)),
            pl.BlockSpec((1, b2.shape[0]), lambda i: (0, 0)),
        ],
        out_specs=pl.BlockSpec((BN, b2.shape[0]), lambda i: (i, 0)),
        out_shape=jax.ShapeDtypeStruct((N, b2.shape[0]), jnp.float32),
    )(agg2, dis, b2r)

    return out


# speedup vs baseline: 1.0107x; 1.0089x over previous
import functools

import jax
import jax.numpy as jnp
from jax import lax
from jax.experimental import pallas as pl
from jax.experimental.pallas import tpu as pltpu
from jax.experimental.pallas import tpu_sc as plsc

N = 10000
NP = 10240
E = 320000
F = 128
H = 64
HS = 32
CP = 16
CHUNK = 128
NC = 2
NS = 16
NW = NC * NS
NBUF = 8
NGRP = 10
CPT = NBUF * NGRP
EP = NW * CPT * CHUNK
RPT = NP // NS
BN = 1000

_mesh = plsc.VectorSubcoreMesh(core_axis_name="c", subcore_axis_name="s")


def _deg_body(dst_hbm, zeros_hbm, ones_hbm, out_hbm, acc, dstb, ones_v, ssem):
    c = lax.axis_index("c")
    s = lax.axis_index("s")
    w = c * NS + s
    pltpu.sync_copy(zeros_hbm.at[pl.ds(s * RPT, RPT)], acc.at[pl.ds(s * RPT, RPT)])
    pltpu.sync_copy(ones_hbm, ones_v)
    pltpu.sync_copy(dst_hbm.at[pl.ds(w * CPT, CPT)], dstb)
    plsc.subcore_barrier()

    def grp(g, carry):
        for b in range(NBUF):
            pltpu.async_copy(ones_v, acc.at[dstb.at[g * NBUF + b]], ssem.at[b], add=True)
        for b in range(NBUF):
            pltpu.make_async_copy(ones_v, acc.at[dstb.at[g * NBUF + b]], ssem.at[b]).wait()
        return carry

    lax.fori_loop(0, NGRP, grp, 0)
    plsc.subcore_barrier()
    pltpu.sync_copy(acc.at[pl.ds(s * RPT, RPT)], out_hbm.at[c].at[pl.ds(s * RPT, RPT)])


def _agg_body(nsplit, d, h_hbm, src_hbm, dst_hbm, zeros_hbm, out_hbm,
              acc, hstage, srcb, dstb, rows, gsem, ssem):
    c = lax.axis_index("c")
    s = lax.axis_index("s")
    w = c * NS + s
    pltpu.sync_copy(src_hbm.at[pl.ds(w * CPT, CPT)], srcb)
    pltpu.sync_copy(dst_hbm.at[pl.ds(w * CPT, CPT)], dstb)

    for kp in range(nsplit):
        @pl.when(c == 0)
        def _():
            pltpu.sync_copy(h_hbm.at[kp].at[pl.ds(s * RPT, RPT)], acc.at[pl.ds(s * RPT, RPT)])

        @pl.when(c != 0)
        def _():
            pltpu.sync_copy(zeros_hbm.at[pl.ds(s * RPT, RPT)], acc.at[pl.ds(s * RPT, RPT)])

        pltpu.sync_copy(h_hbm.at[kp].at[pl.ds(s * RPT, RPT)], hstage.at[pl.ds(s * RPT, RPT)])
        plsc.subcore_barrier()

        for b in range(NBUF):
            pltpu.async_copy(hstage.at[srcb.at[b]], rows.at[b], gsem.at[b])

        def grp(g, carry):
            for b in range(NBUF):
                j = g * NBUF + b
                pltpu.make_async_copy(hstage.at[srcb.at[j]], rows.at[b], gsem.at[b]).wait()
                pltpu.async_copy(rows.at[b], acc.at[dstb.at[j]], ssem.at[b], add=True)
            for b in range(NBUF):
                j = g * NBUF + b
                pltpu.make_async_copy(rows.at[b], acc.at[dstb.at[j]], ssem.at[b]).wait()
                pltpu.async_copy(hstage.at[srcb.at[j + NBUF]], rows.at[b], gsem.at[b])
            return carry

        lax.fori_loop(0, NGRP - 1, grp, 0)
        for b in range(NBUF):
            j = (NGRP - 1) * NBUF + b
            pltpu.make_async_copy(hstage.at[srcb.at[j]], rows.at[b], gsem.at[b]).wait()
            pltpu.async_copy(rows.at[b], acc.at[dstb.at[j]], ssem.at[b], add=True)
        for b in range(NBUF):
            j = (NGRP - 1) * NBUF + b
            pltpu.make_async_copy(rows.at[b], acc.at[dstb.at[j]], ssem.at[b]).wait()
        plsc.subcore_barrier()
        pltpu.sync_copy(acc.at[pl.ds(s * RPT, RPT)],
                        out_hbm.at[c].at[kp].at[pl.ds(s * RPT, RPT)])


def _sc_degree(dst2d, zeros1, ones1):
    return pl.kernel(
        _deg_body,
        out_type=jax.ShapeDtypeStruct((NC, NP, CP), jnp.float32),
        mesh=_mesh,
        scratch_types=[
            pltpu.VMEM_SHARED((NP, CP), jnp.float32),
            pltpu.VMEM((CPT, CHUNK), jnp.int32),
            pltpu.VMEM((CHUNK, CP), jnp.float32),
            pltpu.SemaphoreType.DMA((NBUF,)),
        ],
        compiler_params=pltpu.CompilerParams(use_tc_tiling_on_sc=False),
    )(dst2d, zeros1, ones1)


def _sc_aggregate(nsplit, d, h, src2d, dst2d, zerosd):
    body = functools.partial(_agg_body, nsplit, d)
    return pl.kernel(
        body,
        out_type=jax.ShapeDtypeStruct((NC, nsplit, NP, d), jnp.float32),
        mesh=_mesh,
        scratch_types=[
            pltpu.VMEM_SHARED((NP, d), jnp.float32),
            pltpu.VMEM_SHARED((NP, d), jnp.float32),
            pltpu.VMEM((CPT, CHUNK), jnp.int32),
            pltpu.VMEM((CPT, CHUNK), jnp.int32),
            pltpu.VMEM((NBUF, CHUNK, d), jnp.float32),
            pltpu.SemaphoreType.DMA((NBUF,)),
            pltpu.SemaphoreType.DMA((NBUF,)),
        ],
        compiler_params=pltpu.CompilerParams(use_tc_tiling_on_sc=False),
    )(h, src2d, dst2d, zerosd)


def _tc_pre_body(x_ref, w1_ref, degp_ref, h1s_ref, dis_ref):
    deg = degp_ref[0, :, 0:1] + degp_ref[1, :, 0:1] + 1.0
    dis = lax.rsqrt(deg)
    dis_ref[...] = dis
    res = jnp.dot(x_ref[...], w1_ref[...], preferred_element_type=jnp.float32, precision=lax.Precision.HIGHEST) * dis
    h1s_ref[0] = res[:, :HS]
    h1s_ref[1] = res[:, HS:]


def _tc_mid_body(agg_ref, dis_ref, b1_ref, w2a_ref, w2b_ref, h2s_ref):
    dis = dis_ref[...]
    yl = agg_ref[0, 0] + agg_ref[1, 0]
    yr = agg_ref[0, 1] + agg_ref[1, 1]
    rl = jnp.maximum(dis * yl + b1_ref[:, :HS], 0.0)
    rr = jnp.maximum(dis * yr + b1_ref[:, HS:], 0.0)
    h2 = (jnp.dot(rl, w2a_ref[...], preferred_element_type=jnp.float32, precision=lax.Precision.HIGHEST)
          + jnp.dot(rr, w2b_ref[...], preferred_element_type=jnp.float32, precision=lax.Precision.HIGHEST))
    h2s_ref[...] = h2 * dis


def _tc_post_body(agg_ref, dis_ref, b2_ref, out_ref):
    y2 = agg_ref[0, 0] + agg_ref[1, 0]
    out_ref[...] = dis_ref[...] * y2[:, : out_ref.shape[1]] + b2_ref[...]


def kernel(x, edge_index, W1, b1, W2, b2):
    src32 = edge_index[0].astype(jnp.int32)
    dst32 = edge_index[1].astype(jnp.int32)
    pad = EP - E
    src2d = jnp.concatenate([src32, jnp.zeros((pad,), jnp.int32)]).reshape(EP // CHUNK, CHUNK)
    pad_dst = N + jnp.arange(pad, dtype=jnp.int32) % (NP - N)
    dst2d = jnp.concatenate([dst32, pad_dst]).reshape(EP // CHUNK, CHUNK)
    zeros1 = jnp.zeros((NP, CP), jnp.float32)
    ones1 = jnp.ones((CHUNK, CP), jnp.float32)
    zeros32 = jnp.zeros((NP, HS), jnp.float32)
    zeros16 = jnp.zeros((NP, CP), jnp.float32)
    W2p = jnp.concatenate([W2, jnp.zeros((H, CP - W2.shape[1]), jnp.float32)], axis=1)
    b1r = b1.reshape(1, H)
    b2r = b2.reshape(1, -1)
    nb = N // BN

    degp = _sc_degree(dst2d, zeros1, ones1)

    h1s, dis = pl.pallas_call(
        _tc_pre_body,
        grid=(nb,),
        in_specs=[
            pl.BlockSpec((BN, F), lambda i: (i, 0)),
            pl.BlockSpec((F, H), lambda i: (0, 0)),
            pl.BlockSpec((NC, BN, CP), lambda i: (0, i, 0)),
        ],
        out_specs=[
            pl.BlockSpec((2, BN, HS), lambda i: (0, i, 0)),
            pl.BlockSpec((BN, 1), lambda i: (i, 0)),
        ],
        out_shape=[
            jax.ShapeDtypeStruct((2, NP, HS), jnp.float32),
            jax.ShapeDtypeStruct((N, 1), jnp.float32),
        ],
    )(x, W1, degp)

    agg1 = _sc_aggregate(2, HS, h1s, src2d, dst2d, zeros32)

    h2s = pl.pallas_call(
        _tc_mid_body,
        grid=(nb,),
        in_specs=[
            pl.BlockSpec((NC, 2, BN, HS), lambda i: (0, 0, i, 0)),
            pl.BlockSpec((BN, 1), lambda i: (i, 0)),
            pl.BlockSpec((1, H), lambda i: (0, 0)),
            pl.BlockSpec((HS, CP), lambda i: (0, 0)),
            pl.BlockSpec((HS, CP), lambda i: (0, 0)),
        ],
        out_specs=pl.BlockSpec((BN, CP), lambda i: (i, 0)),
        out_shape=jax.ShapeDtypeStruct((NP, CP), jnp.float32),
    )(agg1, dis, b1r, W2p[:HS], W2p[HS:])

    agg2 = _sc_aggregate(1, CP, h2s.reshape(1, NP, CP), src2d, dst2d, zeros16)

    out = pl.pallas_call(
        _tc_post_body,
        grid=(nb,),
        in_specs=[
            pl.BlockSpec((NC, 1, BN, CP), lambda i: (0, 0, i, 0)),
            pl.BlockSpec((BN, 1), lambda i: (i, 0)),
            pl.BlockSpec((1, b2.shape[0]), lambda i: (0, 0)),
        ],
        out_specs=pl.BlockSpec((BN, b2.shape[0]), lambda i: (i, 0)),
        out_shape=jax.ShapeDtypeStruct((N, b2.shape[0]), jnp.float32),
    )(agg2, dis, b2r)

    return out
